# Initial kernel scaffold; baseline (speedup 1.0000x reference)
import jax
import jax.numpy as jnp
from jax.experimental import pallas as pl


def kernel(node_features, edge_features, senders, receivers, globals_, We1, be1, betae, We2, be2, Wn1, bn1, betan, Wn2, bn2, Wg1, bg1, betag, Wg2, bg2):
    raise NotImplementedError("write your pallas kernel here")


# speedup vs baseline: 2.2886x; 2.2886x over previous
import functools

import jax
import jax.numpy as jnp
from jax import lax
from jax.experimental import pallas as pl
from jax.experimental.pallas import tpu as pltpu
from jax.experimental.pallas import tpu_sc as plsc

N = 10000
E = 320000
D = 128
DE = 16

L = 16
NC = 2
NS = 16
HALF = N // NC
NR = HALF + 8
CH = 64
GR = 2 * CH
DW = D + L
NCHUNKS = E // CH
ZFULL = NR // CH
ZTAIL = NR - ZFULL * CH
WFULL = HALF // CH
WTAIL = HALF - WFULL * CH


_RN = 400
_RE = 4000


def _node_prep_body(nf_ref, ws_ref, wr_ref, be1_ref, t_ref):
    nf = nf_ref[...]
    t_ref[0] = jnp.dot(nf, ws_ref[...], preferred_element_type=jnp.float32)
    t_ref[1] = (jnp.dot(nf, wr_ref[...], preferred_element_type=jnp.float32)
                + be1_ref[...])


def _edge_prep_body(ef_ref, we_ref, o_ref):
    o_ref[...] = jnp.dot(ef_ref[...], we_ref[...],
                         preferred_element_type=jnp.float32)


def _node_prep(nf, ws, wr, be1r):
    return pl.pallas_call(
        _node_prep_body,
        grid=(N // _RN,),
        in_specs=[
            pl.BlockSpec((_RN, D), lambda i: (i, 0)),
            pl.BlockSpec((D, D), lambda i: (0, 0)),
            pl.BlockSpec((D, D), lambda i: (0, 0)),
            pl.BlockSpec((1, D), lambda i: (0, 0)),
        ],
        out_specs=pl.BlockSpec((2, _RN, D), lambda i: (0, i, 0)),
        out_shape=jax.ShapeDtypeStruct((2, N, D), jnp.float32),
    )(nf, ws, wr, be1r)


def _edge_prep(ef, we):
    return pl.pallas_call(
        _edge_prep_body,
        grid=(E // _RE,),
        in_specs=[
            pl.BlockSpec((_RE, DE), lambda i: (i, 0)),
            pl.BlockSpec((DE, D), lambda i: (0, 0)),
        ],
        out_specs=pl.BlockSpec((_RE, D), lambda i: (i, 0)),
        out_shape=jax.ShapeDtypeStruct((E, D), jnp.float32),
    )(ef, we)


_sc_mesh = plsc.VectorSubcoreMesh(core_axis_name="c", subcore_axis_name="s")


@functools.partial(
    pl.kernel,
    out_type=jax.ShapeDtypeStruct((N, D), jnp.float32),
    mesh=_sc_mesh,
    scratch_types=[
        pltpu.VMEM((GR,), jnp.int32),
        pltpu.VMEM((CH,), jnp.int32),
        pltpu.VMEM((GR, D), jnp.float32),
        pltpu.VMEM((CH, D), jnp.float32),
        pltpu.VMEM((CH, D), jnp.float32),
        pltpu.VMEM((D,), jnp.float32),
        pltpu.VMEM_SHARED((NR, D), jnp.float32),
        pltpu.SemaphoreType.DMA,
    ],
)
def _sc_edge_kernel(t_hbm, e1_hbm, s_hbm, r_hbm, nb_hbm,
                    acc_out,
                    cidx, lidx, grows, hrows, erows, nbv,
                    sh_acc, sem):
    cid = lax.axis_index("c")
    sid = lax.axis_index("s")
    lo = cid * HALF

    pltpu.sync_copy(nb_hbm, nbv)
    nb = [nbv[pl.ds(j * L, L)] for j in range(D // L)]

    zero = jnp.zeros((L,), jnp.float32)

    def fill(i, _):
        for j in range(D // L):
            hrows[i, pl.ds(j * L, L)] = zero
        return 0

    lax.fori_loop(0, CH, fill, 0)

    n_zblk = (ZFULL - 1 - sid) // NS + 1

    def zero_blk(k, _):
        b = (sid + k * NS) * CH
        pltpu.sync_copy(hrows, sh_acc.at[pl.ds(b, CH)])
        return 0

    lax.fori_loop(0, n_zblk, zero_blk, 0)

    @pl.when(sid == 0)
    def _zero_tail():
        pltpu.sync_copy(hrows.at[pl.ds(0, ZTAIL)],
                        sh_acc.at[pl.ds(ZFULL * CH, ZTAIL)])

    plsc.subcore_barrier()

    n_my = (NCHUNKS - 1 - sid) // NS + 1

    def chunk_body(k, _):
        base = (sid + k * NS) * CH
        pltpu.sync_copy(r_hbm.at[pl.ds(base, CH)], cidx.at[pl.ds(CH, CH)])
        r_first = cidx[pl.ds(CH, L)][0]
        r_last = cidx[pl.ds(GR - L, L)][L - 1]

        @pl.when((r_last >= lo) & (r_first < lo + HALF))
        def _process():
            pltpu.sync_copy(s_hbm.at[pl.ds(base, CH)], cidx.at[pl.ds(0, CH)])
            for j in range(CH // L):
                sl = pl.ds(CH + j * L, L)
                rv = cidx[sl]
                loc = rv - lo
                bad = (loc < 0) | (loc >= HALF)
                lidx[pl.ds(j * L, L)] = jnp.where(bad, HALF, loc)
                cidx[sl] = rv + N
            dg = pltpu.async_copy(t_hbm.at[cidx], grows, sem)
            de = pltpu.async_copy(e1_hbm.at[pl.ds(base, CH)], erows, sem)
            dg.wait()
            de.wait()

            def row(i, _):
                for j in range(D // L):
                    sl = pl.ds(j * L, L)
                    x = grows[i, sl] + grows[CH + i, sl] + erows[i, sl]
                    hrows[i, sl] = x / (1.0 + jnp.exp(x * nb[j]))
                return 0

            lax.fori_loop(0, CH, row, 0)
            pltpu.sync_copy(hrows, sh_acc.at[lidx], add=True)

        return 0

    lax.fori_loop(0, n_my, chunk_body, 0)
    plsc.subcore_barrier()

    n_wblk = (WFULL - 1 - sid) // NS + 1

    def write_blk(k, _):
        b = (sid + k * NS) * CH
        pltpu.sync_copy(sh_acc.at[pl.ds(b, CH)], hrows)
        pltpu.sync_copy(hrows, acc_out.at[pl.ds(lo + b, CH)])
        return 0

    lax.fori_loop(0, n_wblk, write_blk, 0)

    @pl.when(sid == 0)
    def _write_tail():
        pltpu.sync_copy(sh_acc.at[pl.ds(WFULL * CH, WTAIL)],
                        hrows.at[pl.ds(0, WTAIL)])
        pltpu.sync_copy(hrows.at[pl.ds(0, WTAIL)],
                        acc_out.at[pl.ds(lo + WFULL * CH, WTAIL)])


CHC = 64
NCHC = E // CHC
ZFC = NR // CHC
ZTC = NR - ZFC * CHC
WFC = HALF // CHC
WTC = HALF - WFC * CHC


@functools.partial(
    pl.kernel,
    out_type=jax.ShapeDtypeStruct((N, D), jnp.float32),
    mesh=_sc_mesh,
    scratch_types=[
        pltpu.VMEM((CHC,), jnp.int32),
        pltpu.VMEM((CHC,), jnp.int32),
        pltpu.VMEM((CHC, D), jnp.float32),
        pltpu.VMEM((CHC, D), jnp.float32),
        pltpu.VMEM_SHARED((NR, D), jnp.float32),
    ],
)
def _sc_count_kernel(r_hbm, cnts_out, ridx, lidx, ones_v, zc, sh_cnts):
    cid = lax.axis_index("c")
    sid = lax.axis_index("s")
    lo = cid * HALF

    zero = jnp.zeros((L,), jnp.float32)
    one = jnp.full((L,), 1.0, jnp.float32)

    def fill(i, _):
        for j in range(D // L):
            ones_v[i, pl.ds(j * L, L)] = one
            zc[i, pl.ds(j * L, L)] = zero
        return 0

    lax.fori_loop(0, CHC, fill, 0)

    n_zblk = (ZFC - 1 - sid) // NS + 1

    def zero_blk(k, _):
        b = (sid + k * NS) * CHC
        pltpu.sync_copy(zc, sh_cnts.at[pl.ds(b, CHC)])
        return 0

    lax.fori_loop(0, n_zblk, zero_blk, 0)

    @pl.when(sid == 0)
    def _zero_tail():
        pltpu.sync_copy(zc.at[pl.ds(0, ZTC)],
                        sh_cnts.at[pl.ds(ZFC * CHC, ZTC)])

    plsc.subcore_barrier()

    n_my = (NCHC - 1 - sid) // NS + 1

    def chunk_body(k, _):
        base = (sid + k * NS) * CHC
        pltpu.sync_copy(r_hbm.at[pl.ds(base, CHC)], ridx)
        r_first = ridx[pl.ds(0, L)][0]
        r_last = ridx[pl.ds(CHC - L, L)][L - 1]

        @pl.when((r_last >= lo) & (r_first < lo + HALF))
        def _process():
            for j in range(CHC // L):
                sl = pl.ds(j * L, L)
                rv = ridx[sl]
                loc = rv - lo
                bad = (loc < 0) | (loc >= HALF)
                lidx[sl] = jnp.where(bad, HALF, loc)
            pltpu.sync_copy(ones_v, sh_cnts.at[lidx], add=True)

        return 0

    lax.fori_loop(0, n_my, chunk_body, 0)
    plsc.subcore_barrier()

    n_wblk = (WFC - 1 - sid) // NS + 1

    def write_blk(k, _):
        b = (sid + k * NS) * CHC
        pltpu.sync_copy(sh_cnts.at[pl.ds(b, CHC)], zc)
        pltpu.sync_copy(zc, cnts_out.at[pl.ds(lo + b, CHC)])
        return 0

    lax.fori_loop(0, n_wblk, write_blk, 0)

    @pl.when(sid == 0)
    def _write_tail():
        pltpu.sync_copy(sh_cnts.at[pl.ds(WFC * CHC, WTC)],
                        zc.at[pl.ds(0, WTC)])
        pltpu.sync_copy(zc.at[pl.ds(0, WTC)],
                        cnts_out.at[pl.ds(lo + WFC * CHC, WTC)])


_NBLK = N // _RN


def _finish_body(s_ref, c_ref, nf_ref, g_ref,
                 we2_ref, be2_ref,
                 wn1a_ref, wn1b_ref, bn1_ref, nbn_ref, wn2_ref, bn2_ref,
                 wg1g_ref, wg1n_ref, wg1m_ref, bg1_ref, nbg_ref,
                 wg2_ref, bg2_ref,
                 out_ref, gout_ref, acc_no_ref, acc_h_ref):
    i = pl.program_id(0)

    s = s_ref[...]
    cnt = c_ref[...][:, :1]
    mean_h = s / jnp.maximum(cnt, 1.0)
    ind = (cnt > 0.0).astype(jnp.float32)
    agg = (jnp.dot(mean_h, we2_ref[...], preferred_element_type=jnp.float32)
           + ind * be2_ref[...])

    nf = nf_ref[...]
    z = (jnp.dot(nf, wn1a_ref[...], preferred_element_type=jnp.float32)
         + jnp.dot(agg, wn1b_ref[...], preferred_element_type=jnp.float32)
         + bn1_ref[...])
    hid = z / (1.0 + jnp.exp(z * nbn_ref[...]))
    no = nf + jnp.dot(hid, wn2_ref[...],
                      preferred_element_type=jnp.float32) + bn2_ref[...]
    out_ref[...] = no

    @pl.when(i == 0)
    def _init():
        acc_no_ref[0:1, :] = jnp.zeros((1, D), jnp.float32)
        acc_h_ref[0:1, :] = jnp.zeros((1, D), jnp.float32)

    acc_no_ref[0:1, :] += jnp.sum(no, axis=0, keepdims=True)
    acc_h_ref[0:1, :] += jnp.sum(s, axis=0, keepdims=True)

    @pl.when(i == _NBLK - 1)
    def _globals():
        mean_no = acc_no_ref[0:1, :] * (1.0 / N)
        mean_em = (jnp.dot(acc_h_ref[0:1, :] * (1.0 / E), we2_ref[...],
                           preferred_element_type=jnp.float32) + be2_ref[...])
        g = g_ref[...]
        zg = (jnp.dot(g, wg1g_ref[...], preferred_element_type=jnp.float32)
              + jnp.dot(mean_no, wg1n_ref[...], preferred_element_type=jnp.float32)
              + jnp.dot(mean_em, wg1m_ref[...], preferred_element_type=jnp.float32)
              + bg1_ref[...])
        gh = zg / (1.0 + jnp.exp(zg * nbg_ref[...]))
        gout_ref[...] = g + jnp.dot(gh, wg2_ref[...],
                                    preferred_element_type=jnp.float32) + bg2_ref[...]


def _finish(sums, cnts, nf, g, we2, be2, wn1a, wn1b, bn1, nbn, wn2, bn2,
            wg1g, wg1n, wg1m, bg1, nbg, wg2, bg2):
    full = lambda shape: pl.BlockSpec(shape, lambda i: tuple(0 for _ in shape))
    row = lambda shape: pl.BlockSpec(shape, lambda i: (i, 0))
    return pl.pallas_call(
        _finish_body,
        grid=(_NBLK,),
        in_specs=[
            row((_RN, D)),
            row((_RN, D)),
            row((_RN, D)),
            full((1, D)),
            full((D, D)), full((1, D)),
            full((D, D)), full((D, D)), full((1, D)), full((1, D)),
            full((D, D)), full((1, D)),
            full((D, D)), full((D, D)), full((D, D)), full((1, D)), full((1, D)),
            full((D, D)), full((1, D)),
        ],
        out_specs=[row((_RN, D)), full((1, D))],
        out_shape=[jax.ShapeDtypeStruct((N, D), jnp.float32),
                   jax.ShapeDtypeStruct((1, D), jnp.float32)],
        scratch_shapes=[pltpu.VMEM((8, D), jnp.float32),
                        pltpu.VMEM((8, D), jnp.float32)],
    )(sums, cnts, nf, g, we2, be2, wn1a, wn1b, bn1, nbn, wn2, bn2,
      wg1g, wg1n, wg1m, bg1, nbg, wg2, bg2)


def kernel(node_features, edge_features, senders, receivers, globals_,
           We1, be1, betae, We2, be2,
           Wn1, bn1, betan, Wn2, bn2,
           Wg1, bg1, betag, Wg2, bg2):
    tabs = _node_prep(node_features, We1[DE:DE + D], We1[DE + D:],
                      be1[None, :])
    table = tabs.reshape(2 * N, D)
    ef1 = _edge_prep(edge_features, We1[:DE])

    r32 = receivers.astype(jnp.int32)
    sums = _sc_edge_kernel(
        table, ef1, senders.astype(jnp.int32), r32, -betae)
    cnts = _sc_count_kernel(r32)

    node_out, globals_out = _finish(
        sums, cnts,
        node_features, globals_,
        We2, be2[None, :],
        Wn1[:D], Wn1[D:], bn1[None, :], -betan[None, :], Wn2, bn2[None, :],
        Wg1[:D], Wg1[D:2 * D], Wg1[2 * D:], bg1[None, :], -betag[None, :],
        Wg2, bg2[None, :])

    return (node_out, edge_features, globals_out)


# speedup vs baseline: 2.9625x; 1.2945x over previous
import functools

import jax
import jax.numpy as jnp
from jax import lax
from jax.experimental import pallas as pl
from jax.experimental.pallas import tpu as pltpu
from jax.experimental.pallas import tpu_sc as plsc

N = 10000
E = 320000
D = 128
DE = 16

L = 16
NC = 2
NS = 16
HALF = N // NC
NR = HALF + 8
CH = 128
GR = 2 * CH
DW = D + L
NCHUNKS = E // CH
ZFULL = NR // CH
ZTAIL = NR - ZFULL * CH
WFULL = HALF // CH
WTAIL = HALF - WFULL * CH


_RN = 400
_RE = 4000


def _node_prep_body(nf_ref, ws_ref, wr_ref, be1_ref, t_ref):
    nf = nf_ref[...]
    t_ref[0] = jnp.dot(nf, ws_ref[...], preferred_element_type=jnp.float32)
    t_ref[1] = (jnp.dot(nf, wr_ref[...], preferred_element_type=jnp.float32)
                + be1_ref[...])


def _edge_prep_body(ef_ref, we_ref, o_ref):
    o_ref[...] = jnp.dot(ef_ref[...], we_ref[...],
                         preferred_element_type=jnp.float32)


def _node_prep(nf, ws, wr, be1r):
    return pl.pallas_call(
        _node_prep_body,
        grid=(N // _RN,),
        in_specs=[
            pl.BlockSpec((_RN, D), lambda i: (i, 0)),
            pl.BlockSpec((D, D), lambda i: (0, 0)),
            pl.BlockSpec((D, D), lambda i: (0, 0)),
            pl.BlockSpec((1, D), lambda i: (0, 0)),
        ],
        out_specs=pl.BlockSpec((2, _RN, D), lambda i: (0, i, 0)),
        out_shape=jax.ShapeDtypeStruct((2, N, D), jnp.float32),
    )(nf, ws, wr, be1r)


def _edge_prep(ef, we):
    return pl.pallas_call(
        _edge_prep_body,
        grid=(E // _RE,),
        in_specs=[
            pl.BlockSpec((_RE, DE), lambda i: (i, 0)),
            pl.BlockSpec((DE, D), lambda i: (0, 0)),
        ],
        out_specs=pl.BlockSpec((_RE, D), lambda i: (i, 0)),
        out_shape=jax.ShapeDtypeStruct((E, D), jnp.float32),
    )(ef, we)


_sc_mesh = plsc.VectorSubcoreMesh(core_axis_name="c", subcore_axis_name="s")


@functools.partial(
    pl.kernel,
    out_type=jax.ShapeDtypeStruct((N, D), jnp.float32),
    mesh=_sc_mesh,
    scratch_types=[
        pltpu.VMEM((GR,), jnp.int32),
        pltpu.VMEM((CH,), jnp.int32),
        pltpu.VMEM((GR, D), jnp.float32),
        pltpu.VMEM((CH, D), jnp.float32),
        pltpu.VMEM((CH, D), jnp.float32),
        pltpu.VMEM((D,), jnp.float32),
        pltpu.VMEM_SHARED((NR, D), jnp.float32),
        pltpu.SemaphoreType.DMA,
    ],
)
def _sc_edge_kernel(t_hbm, e1_hbm, s_hbm, r_hbm, nb_hbm,
                    acc_out,
                    cidx, lidx, grows, hrows, erows, nbv,
                    sh_acc, sem):
    cid = lax.axis_index("c")
    sid = lax.axis_index("s")
    lo = cid * HALF

    pltpu.sync_copy(nb_hbm, nbv)
    nb = [nbv[pl.ds(j * L, L)] for j in range(D // L)]

    zero = jnp.zeros((L,), jnp.float32)

    def fill(i, _):
        for j in range(D // L):
            hrows[i, pl.ds(j * L, L)] = zero
        return 0

    lax.fori_loop(0, CH, fill, 0)

    n_zblk = (ZFULL - 1 - sid) // NS + 1

    def zero_blk(k, _):
        b = (sid + k * NS) * CH
        pltpu.sync_copy(hrows, sh_acc.at[pl.ds(b, CH)])
        return 0

    lax.fori_loop(0, n_zblk, zero_blk, 0)

    @pl.when(sid == 0)
    def _zero_tail():
        pltpu.sync_copy(hrows.at[pl.ds(0, ZTAIL)],
                        sh_acc.at[pl.ds(ZFULL * CH, ZTAIL)])

    plsc.subcore_barrier()

    n_my = (NCHUNKS - 1 - sid) // NS + 1

    def chunk_body(k, _):
        base = (sid + k * NS) * CH
        pltpu.sync_copy(r_hbm.at[pl.ds(base, CH)], cidx.at[pl.ds(CH, CH)])
        r_first = cidx[pl.ds(CH, L)][0]
        r_last = cidx[pl.ds(GR - L, L)][L - 1]

        @pl.when((r_last >= lo) & (r_first < lo + HALF))
        def _process():
            pltpu.sync_copy(s_hbm.at[pl.ds(base, CH)], cidx.at[pl.ds(0, CH)])
            for j in range(CH // L):
                sl = pl.ds(CH + j * L, L)
                rv = cidx[sl]
                loc = rv - lo
                bad = (loc < 0) | (loc >= HALF)
                lidx[pl.ds(j * L, L)] = jnp.where(bad, HALF, loc)
                cidx[sl] = rv + N
            dg1 = pltpu.async_copy(t_hbm.at[cidx.at[pl.ds(0, CH)]],
                                   grows.at[pl.ds(0, CH)], sem)
            dg2 = pltpu.async_copy(t_hbm.at[cidx.at[pl.ds(CH, CH)]],
                                   grows.at[pl.ds(CH, CH)], sem)
            de = pltpu.async_copy(e1_hbm.at[pl.ds(base, CH)], erows, sem)
            dg1.wait()
            dg2.wait()
            de.wait()

            def row(i, _):
                for j in range(D // L):
                    sl = pl.ds(j * L, L)
                    x = grows[i, sl] + grows[CH + i, sl] + erows[i, sl]
                    hrows[i, sl] = x / (1.0 + jnp.exp(x * nb[j]))
                return 0

            lax.fori_loop(0, CH, row, 0)
            pltpu.sync_copy(hrows, sh_acc.at[lidx], add=True)

        return 0

    lax.fori_loop(0, n_my, chunk_body, 0)
    plsc.subcore_barrier()

    n_wblk = (WFULL - 1 - sid) // NS + 1

    def write_blk(k, _):
        b = (sid + k * NS) * CH
        pltpu.sync_copy(sh_acc.at[pl.ds(b, CH)], hrows)
        pltpu.sync_copy(hrows, acc_out.at[pl.ds(lo + b, CH)])
        return 0

    lax.fori_loop(0, n_wblk, write_blk, 0)

    @pl.when(sid == 0)
    def _write_tail():
        pltpu.sync_copy(sh_acc.at[pl.ds(WFULL * CH, WTAIL)],
                        hrows.at[pl.ds(0, WTAIL)])
        pltpu.sync_copy(hrows.at[pl.ds(0, WTAIL)],
                        acc_out.at[pl.ds(lo + WFULL * CH, WTAIL)])


CHC = 128
NCHC = E // CHC
ZFC = NR // CHC
ZTC = NR - ZFC * CHC
WFC = HALF // CHC
WTC = HALF - WFC * CHC


@functools.partial(
    pl.kernel,
    out_type=jax.ShapeDtypeStruct((N, D), jnp.float32),
    mesh=_sc_mesh,
    scratch_types=[
        pltpu.VMEM((CHC,), jnp.int32),
        pltpu.VMEM((CHC,), jnp.int32),
        pltpu.VMEM((CHC, D), jnp.float32),
        pltpu.VMEM((CHC, D), jnp.float32),
        pltpu.VMEM_SHARED((NR, D), jnp.float32),
    ],
)
def _sc_count_kernel(r_hbm, cnts_out, ridx, lidx, ones_v, zc, sh_cnts):
    cid = lax.axis_index("c")
    sid = lax.axis_index("s")
    lo = cid * HALF

    zero = jnp.zeros((L,), jnp.float32)
    one = jnp.full((L,), 1.0, jnp.float32)

    def fill(i, _):
        for j in range(D // L):
            ones_v[i, pl.ds(j * L, L)] = one
            zc[i, pl.ds(j * L, L)] = zero
        return 0

    lax.fori_loop(0, CHC, fill, 0)

    n_zblk = (ZFC - 1 - sid) // NS + 1

    def zero_blk(k, _):
        b = (sid + k * NS) * CHC
        pltpu.sync_copy(zc, sh_cnts.at[pl.ds(b, CHC)])
        return 0

    lax.fori_loop(0, n_zblk, zero_blk, 0)

    @pl.when(sid == 0)
    def _zero_tail():
        pltpu.sync_copy(zc.at[pl.ds(0, ZTC)],
                        sh_cnts.at[pl.ds(ZFC * CHC, ZTC)])

    plsc.subcore_barrier()

    n_my = (NCHC - 1 - sid) // NS + 1

    def chunk_body(k, _):
        base = (sid + k * NS) * CHC
        pltpu.sync_copy(r_hbm.at[pl.ds(base, CHC)], ridx)
        r_first = ridx[pl.ds(0, L)][0]
        r_last = ridx[pl.ds(CHC - L, L)][L - 1]

        @pl.when((r_last >= lo) & (r_first < lo + HALF))
        def _process():
            for j in range(CHC // L):
                sl = pl.ds(j * L, L)
                rv = ridx[sl]
                loc = rv - lo
                bad = (loc < 0) | (loc >= HALF)
                lidx[sl] = jnp.where(bad, HALF, loc)
            pltpu.sync_copy(ones_v, sh_cnts.at[lidx], add=True)

        return 0

    lax.fori_loop(0, n_my, chunk_body, 0)
    plsc.subcore_barrier()

    n_wblk = (WFC - 1 - sid) // NS + 1

    def write_blk(k, _):
        b = (sid + k * NS) * CHC
        pltpu.sync_copy(sh_cnts.at[pl.ds(b, CHC)], zc)
        pltpu.sync_copy(zc, cnts_out.at[pl.ds(lo + b, CHC)])
        return 0

    lax.fori_loop(0, n_wblk, write_blk, 0)

    @pl.when(sid == 0)
    def _write_tail():
        pltpu.sync_copy(sh_cnts.at[pl.ds(WFC * CHC, WTC)],
                        zc.at[pl.ds(0, WTC)])
        pltpu.sync_copy(zc.at[pl.ds(0, WTC)],
                        cnts_out.at[pl.ds(lo + WFC * CHC, WTC)])


_NBLK = N // _RN


def _finish_body(s_ref, c_ref, nf_ref, g_ref,
                 we2_ref, be2_ref,
                 wn1a_ref, wn1b_ref, bn1_ref, nbn_ref, wn2_ref, bn2_ref,
                 wg1g_ref, wg1n_ref, wg1m_ref, bg1_ref, nbg_ref,
                 wg2_ref, bg2_ref,
                 out_ref, gout_ref, acc_no_ref, acc_h_ref):
    i = pl.program_id(0)

    s = s_ref[...]
    cnt = c_ref[...][:, :1]
    mean_h = s / jnp.maximum(cnt, 1.0)
    ind = (cnt > 0.0).astype(jnp.float32)
    agg = (jnp.dot(mean_h, we2_ref[...], preferred_element_type=jnp.float32)
           + ind * be2_ref[...])

    nf = nf_ref[...]
    z = (jnp.dot(nf, wn1a_ref[...], preferred_element_type=jnp.float32)
         + jnp.dot(agg, wn1b_ref[...], preferred_element_type=jnp.float32)
         + bn1_ref[...])
    hid = z / (1.0 + jnp.exp(z * nbn_ref[...]))
    no = nf + jnp.dot(hid, wn2_ref[...],
                      preferred_element_type=jnp.float32) + bn2_ref[...]
    out_ref[...] = no

    @pl.when(i == 0)
    def _init():
        acc_no_ref[0:1, :] = jnp.zeros((1, D), jnp.float32)
        acc_h_ref[0:1, :] = jnp.zeros((1, D), jnp.float32)

    acc_no_ref[0:1, :] += jnp.sum(no, axis=0, keepdims=True)
    acc_h_ref[0:1, :] += jnp.sum(s, axis=0, keepdims=True)

    @pl.when(i == _NBLK - 1)
    def _globals():
        mean_no = acc_no_ref[0:1, :] * (1.0 / N)
        mean_em = (jnp.dot(acc_h_ref[0:1, :] * (1.0 / E), we2_ref[...],
                           preferred_element_type=jnp.float32) + be2_ref[...])
        g = g_ref[...]
        zg = (jnp.dot(g, wg1g_ref[...], preferred_element_type=jnp.float32)
              + jnp.dot(mean_no, wg1n_ref[...], preferred_element_type=jnp.float32)
              + jnp.dot(mean_em, wg1m_ref[...], preferred_element_type=jnp.float32)
              + bg1_ref[...])
        gh = zg / (1.0 + jnp.exp(zg * nbg_ref[...]))
        gout_ref[...] = g + jnp.dot(gh, wg2_ref[...],
                                    preferred_element_type=jnp.float32) + bg2_ref[...]


def _finish(sums, cnts, nf, g, we2, be2, wn1a, wn1b, bn1, nbn, wn2, bn2,
            wg1g, wg1n, wg1m, bg1, nbg, wg2, bg2):
    full = lambda shape: pl.BlockSpec(shape, lambda i: tuple(0 for _ in shape))
    row = lambda shape: pl.BlockSpec(shape, lambda i: (i, 0))
    return pl.pallas_call(
        _finish_body,
        grid=(_NBLK,),
        in_specs=[
            row((_RN, D)),
            row((_RN, D)),
            row((_RN, D)),
            full((1, D)),
            full((D, D)), full((1, D)),
            full((D, D)), full((D, D)), full((1, D)), full((1, D)),
            full((D, D)), full((1, D)),
            full((D, D)), full((D, D)), full((D, D)), full((1, D)), full((1, D)),
            full((D, D)), full((1, D)),
        ],
        out_specs=[row((_RN, D)), full((1, D))],
        out_shape=[jax.ShapeDtypeStruct((N, D), jnp.float32),
                   jax.ShapeDtypeStruct((1, D), jnp.float32)],
        scratch_shapes=[pltpu.VMEM((8, D), jnp.float32),
                        pltpu.VMEM((8, D), jnp.float32)],
    )(sums, cnts, nf, g, we2, be2, wn1a, wn1b, bn1, nbn, wn2, bn2,
      wg1g, wg1n, wg1m, bg1, nbg, wg2, bg2)


def kernel(node_features, edge_features, senders, receivers, globals_,
           We1, be1, betae, We2, be2,
           Wn1, bn1, betan, Wn2, bn2,
           Wg1, bg1, betag, Wg2, bg2):
    tabs = _node_prep(node_features, We1[DE:DE + D], We1[DE + D:],
                      be1[None, :])
    table = tabs.reshape(2 * N, D)
    ef1 = _edge_prep(edge_features, We1[:DE])

    r32 = receivers.astype(jnp.int32)
    sums = _sc_edge_kernel(
        table, ef1, senders.astype(jnp.int32), r32, -betae)
    cnts = _sc_count_kernel(r32)

    node_out, globals_out = _finish(
        sums, cnts,
        node_features, globals_,
        We2, be2[None, :],
        Wn1[:D], Wn1[D:], bn1[None, :], -betan[None, :], Wn2, bn2[None, :],
        Wg1[:D], Wg1[D:2 * D], Wg1[2 * D:], bg1[None, :], -betag[None, :],
        Wg2, bg2[None, :])

    return (node_out, edge_features, globals_out)


# speedup vs baseline: 3.4119x; 1.1517x over previous
import functools

import jax
import jax.numpy as jnp
from jax import lax
from jax.experimental import pallas as pl
from jax.experimental.pallas import tpu as pltpu
from jax.experimental.pallas import tpu_sc as plsc

N = 10000
E = 320000
D = 128
DE = 16

L = 16
NC = 2
NS = 16
HALF = N // NC
NR = HALF + 8
CH = 64
GR = 2 * CH
DW = D + L
NCHUNKS = E // CH
ZFULL = NR // CH
ZTAIL = NR - ZFULL * CH
WFULL = HALF // CH
WTAIL = HALF - WFULL * CH


_RN = 400
_RE = 4000


def _node_prep_body(nf_ref, ws_ref, wr_ref, be1_ref, t_ref):
    nf = nf_ref[...]
    t_ref[0] = jnp.dot(nf, ws_ref[...], preferred_element_type=jnp.float32)
    t_ref[1] = (jnp.dot(nf, wr_ref[...], preferred_element_type=jnp.float32)
                + be1_ref[...])


def _edge_prep_body(ef_ref, we_ref, o_ref):
    o_ref[...] = jnp.dot(ef_ref[...], we_ref[...],
                         preferred_element_type=jnp.float32)


def _node_prep(nf, ws, wr, be1r):
    return pl.pallas_call(
        _node_prep_body,
        grid=(N // _RN,),
        in_specs=[
            pl.BlockSpec((_RN, D), lambda i: (i, 0)),
            pl.BlockSpec((D, D), lambda i: (0, 0)),
            pl.BlockSpec((D, D), lambda i: (0, 0)),
            pl.BlockSpec((1, D), lambda i: (0, 0)),
        ],
        out_specs=pl.BlockSpec((2, _RN, D), lambda i: (0, i, 0)),
        out_shape=jax.ShapeDtypeStruct((2, N, D), jnp.float32),
    )(nf, ws, wr, be1r)


def _edge_prep(ef, we):
    return pl.pallas_call(
        _edge_prep_body,
        grid=(E // _RE,),
        in_specs=[
            pl.BlockSpec((_RE, DE), lambda i: (i, 0)),
            pl.BlockSpec((DE, D), lambda i: (0, 0)),
        ],
        out_specs=pl.BlockSpec((_RE, D), lambda i: (i, 0)),
        out_shape=jax.ShapeDtypeStruct((E, D), jnp.float32),
    )(ef, we)


_sc_mesh = plsc.VectorSubcoreMesh(core_axis_name="c", subcore_axis_name="s")


@functools.partial(
    pl.kernel,
    out_type=jax.ShapeDtypeStruct((N, D), jnp.float32),
    mesh=_sc_mesh,
    scratch_types=[
        pltpu.VMEM((2 * GR,), jnp.int32),
        pltpu.VMEM((2, CH), jnp.int32),
        pltpu.VMEM((2 * GR, D), jnp.float32),
        pltpu.VMEM((CH, D), jnp.float32),
        pltpu.VMEM((2 * CH, D), jnp.float32),
        pltpu.VMEM((D,), jnp.float32),
        pltpu.VMEM_SHARED((NR, D), jnp.float32),
        pltpu.SemaphoreType.DMA,
        pltpu.SemaphoreType.DMA,
        pltpu.SemaphoreType.DMA,
        pltpu.SemaphoreType.DMA,
        pltpu.SemaphoreType.DMA,
    ],
)
def _sc_edge_kernel(t_hbm, e1_hbm, s_hbm, r_hbm, nb_hbm,
                    acc_out,
                    cidx, lidx, grows, hrows, erows, nbv,
                    sh_acc, semg0, semg1, seme0, seme1, semsc):
    cid = lax.axis_index("c")
    sid = lax.axis_index("s")
    lo = cid * HALF

    pltpu.sync_copy(nb_hbm, nbv)
    nb = [nbv[pl.ds(j * L, L)] for j in range(D // L)]

    zero = jnp.zeros((L,), jnp.float32)

    def fill(i, _):
        for j in range(D // L):
            hrows[i, pl.ds(j * L, L)] = zero
        return 0

    lax.fori_loop(0, CH, fill, 0)

    n_zblk = (ZFULL - 1 - sid) // NS + 1

    def zero_blk(k, _):
        b = (sid + k * NS) * CH
        pltpu.sync_copy(hrows, sh_acc.at[pl.ds(b, CH)])
        return 0

    lax.fori_loop(0, n_zblk, zero_blk, 0)

    @pl.when(sid == 0)
    def _zero_tail():
        pltpu.sync_copy(hrows.at[pl.ds(0, ZTAIL)],
                        sh_acc.at[pl.ds(ZFULL * CH, ZTAIL)])

    plsc.subcore_barrier()

    n_my = (NCHUNKS - 1 - sid) // NS + 1
    hi = lo + HALF

    def prefetch(kv, slot, semg, seme):
        kc = jnp.minimum(kv, n_my - 1)
        base = (sid + kc * NS) * CH
        off = slot * GR
        pltpu.sync_copy(r_hbm.at[pl.ds(base, CH)], cidx.at[pl.ds(off + CH, CH)])
        r_first = cidx[pl.ds(off + CH, L)][0]
        r_last = cidx[pl.ds(off + GR - L, L)][L - 1]
        do = (kv < n_my) & (r_last >= lo) & (r_first < hi)

        @pl.when(do)
        def _():
            pltpu.sync_copy(s_hbm.at[pl.ds(base, CH)], cidx.at[pl.ds(off, CH)])
            for j in range(CH // L):
                sl = pl.ds(off + CH + j * L, L)
                rv = cidx[sl]
                loc = rv - lo
                bad = (loc < 0) | (loc >= HALF)
                lidx[slot, pl.ds(j * L, L)] = jnp.where(bad, HALF, loc)
                cidx[sl] = rv + N
            pltpu.async_copy(t_hbm.at[cidx.at[pl.ds(off, CH)]],
                             grows.at[pl.ds(off, CH)], semg)
            pltpu.async_copy(t_hbm.at[cidx.at[pl.ds(off + CH, CH)]],
                             grows.at[pl.ds(off + CH, CH)], semg)
            pltpu.async_copy(e1_hbm.at[pl.ds(base, CH)],
                             erows.at[pl.ds(slot * CH, CH)], seme)

        return do

    def compute(slot, semg, seme, do_c, sc_pend):
        @pl.when(do_c)
        def _():
            @pl.when(sc_pend)
            def _drain():
                pltpu.make_async_copy(
                    hrows, sh_acc.at[lidx.at[1 - slot]], semsc).wait()

            off = slot * GR
            pltpu.make_async_copy(t_hbm.at[cidx.at[pl.ds(off, CH)]],
                                  grows.at[pl.ds(off, CH)], semg).wait()
            pltpu.make_async_copy(t_hbm.at[cidx.at[pl.ds(off + CH, CH)]],
                                  grows.at[pl.ds(off + CH, CH)], semg).wait()
            pltpu.make_async_copy(e1_hbm.at[pl.ds(0, CH)],
                                  erows.at[pl.ds(slot * CH, CH)], seme).wait()

            eoff = slot * CH

            def row(i, _):
                for j in range(D // L):
                    sl = pl.ds(j * L, L)
                    x = (grows[off + i, sl] + grows[off + CH + i, sl]
                         + erows[eoff + i, sl])
                    hrows[i, sl] = x / (1.0 + jnp.exp(x * nb[j]))
                return 0

            lax.fori_loop(0, CH, row, 0)
            pltpu.async_copy(hrows, sh_acc.at[lidx.at[slot]], semsc, add=True)

    def pair(m, carry):
        do_prev, sc_pend = carry
        do_a = prefetch(2 * m, 0, semg0, seme0)
        compute(1, semg1, seme1, do_prev, sc_pend)
        sc1 = do_prev | sc_pend
        do_b = prefetch(2 * m + 1, 1, semg1, seme1)
        compute(0, semg0, seme0, do_a, sc1)
        return do_b, do_a | sc1

    n_pairs = n_my // 2 + 1
    do_last, sc_last = lax.fori_loop(
        0, n_pairs, pair, (jnp.bool_(False), jnp.bool_(False)))
    compute(1, semg1, seme1, do_last, sc_last)

    @pl.when(do_last | sc_last)
    def _final_drain():
        pltpu.make_async_copy(hrows, sh_acc.at[lidx.at[0]], semsc).wait()

    plsc.subcore_barrier()

    n_wblk = (WFULL - 1 - sid) // NS + 1

    def write_blk(k, _):
        b = (sid + k * NS) * CH
        pltpu.sync_copy(sh_acc.at[pl.ds(b, CH)], hrows)
        pltpu.sync_copy(hrows, acc_out.at[pl.ds(lo + b, CH)])
        return 0

    lax.fori_loop(0, n_wblk, write_blk, 0)

    @pl.when(sid == 0)
    def _write_tail():
        pltpu.sync_copy(sh_acc.at[pl.ds(WFULL * CH, WTAIL)],
                        hrows.at[pl.ds(0, WTAIL)])
        pltpu.sync_copy(hrows.at[pl.ds(0, WTAIL)],
                        acc_out.at[pl.ds(lo + WFULL * CH, WTAIL)])


CHC = 128
NCHC = E // CHC
ZFC = NR // CHC
ZTC = NR - ZFC * CHC
WFC = HALF // CHC
WTC = HALF - WFC * CHC


@functools.partial(
    pl.kernel,
    out_type=jax.ShapeDtypeStruct((N, D), jnp.float32),
    mesh=_sc_mesh,
    scratch_types=[
        pltpu.VMEM((CHC,), jnp.int32),
        pltpu.VMEM((CHC,), jnp.int32),
        pltpu.VMEM((CHC, D), jnp.float32),
        pltpu.VMEM((CHC, D), jnp.float32),
        pltpu.VMEM_SHARED((NR, D), jnp.float32),
    ],
)
def _sc_count_kernel(r_hbm, cnts_out, ridx, lidx, ones_v, zc, sh_cnts):
    cid = lax.axis_index("c")
    sid = lax.axis_index("s")
    lo = cid * HALF

    zero = jnp.zeros((L,), jnp.float32)
    one = jnp.full((L,), 1.0, jnp.float32)

    def fill(i, _):
        for j in range(D // L):
            ones_v[i, pl.ds(j * L, L)] = one
            zc[i, pl.ds(j * L, L)] = zero
        return 0

    lax.fori_loop(0, CHC, fill, 0)

    n_zblk = (ZFC - 1 - sid) // NS + 1

    def zero_blk(k, _):
        b = (sid + k * NS) * CHC
        pltpu.sync_copy(zc, sh_cnts.at[pl.ds(b, CHC)])
        return 0

    lax.fori_loop(0, n_zblk, zero_blk, 0)

    @pl.when(sid == 0)
    def _zero_tail():
        pltpu.sync_copy(zc.at[pl.ds(0, ZTC)],
                        sh_cnts.at[pl.ds(ZFC * CHC, ZTC)])

    plsc.subcore_barrier()

    n_my = (NCHC - 1 - sid) // NS + 1

    def chunk_body(k, _):
        base = (sid + k * NS) * CHC
        pltpu.sync_copy(r_hbm.at[pl.ds(base, CHC)], ridx)
        r_first = ridx[pl.ds(0, L)][0]
        r_last = ridx[pl.ds(CHC - L, L)][L - 1]

        @pl.when((r_last >= lo) & (r_first < lo + HALF))
        def _process():
            for j in range(CHC // L):
                sl = pl.ds(j * L, L)
                rv = ridx[sl]
                loc = rv - lo
                bad = (loc < 0) | (loc >= HALF)
                lidx[sl] = jnp.where(bad, HALF, loc)
            pltpu.sync_copy(ones_v, sh_cnts.at[lidx], add=True)

        return 0

    lax.fori_loop(0, n_my, chunk_body, 0)
    plsc.subcore_barrier()

    n_wblk = (WFC - 1 - sid) // NS + 1

    def write_blk(k, _):
        b = (sid + k * NS) * CHC
        pltpu.sync_copy(sh_cnts.at[pl.ds(b, CHC)], zc)
        pltpu.sync_copy(zc, cnts_out.at[pl.ds(lo + b, CHC)])
        return 0

    lax.fori_loop(0, n_wblk, write_blk, 0)

    @pl.when(sid == 0)
    def _write_tail():
        pltpu.sync_copy(sh_cnts.at[pl.ds(WFC * CHC, WTC)],
                        zc.at[pl.ds(0, WTC)])
        pltpu.sync_copy(zc.at[pl.ds(0, WTC)],
                        cnts_out.at[pl.ds(lo + WFC * CHC, WTC)])


_NBLK = N // _RN


def _finish_body(s_ref, c_ref, nf_ref, g_ref,
                 we2_ref, be2_ref,
                 wn1a_ref, wn1b_ref, bn1_ref, nbn_ref, wn2_ref, bn2_ref,
                 wg1g_ref, wg1n_ref, wg1m_ref, bg1_ref, nbg_ref,
                 wg2_ref, bg2_ref,
                 out_ref, gout_ref, acc_no_ref, acc_h_ref):
    i = pl.program_id(0)

    s = s_ref[...]
    cnt = c_ref[...][:, :1]
    mean_h = s / jnp.maximum(cnt, 1.0)
    ind = (cnt > 0.0).astype(jnp.float32)
    agg = (jnp.dot(mean_h, we2_ref[...], preferred_element_type=jnp.float32)
           + ind * be2_ref[...])

    nf = nf_ref[...]
    z = (jnp.dot(nf, wn1a_ref[...], preferred_element_type=jnp.float32)
         + jnp.dot(agg, wn1b_ref[...], preferred_element_type=jnp.float32)
         + bn1_ref[...])
    hid = z / (1.0 + jnp.exp(z * nbn_ref[...]))
    no = nf + jnp.dot(hid, wn2_ref[...],
                      preferred_element_type=jnp.float32) + bn2_ref[...]
    out_ref[...] = no

    @pl.when(i == 0)
    def _init():
        acc_no_ref[0:1, :] = jnp.zeros((1, D), jnp.float32)
        acc_h_ref[0:1, :] = jnp.zeros((1, D), jnp.float32)

    acc_no_ref[0:1, :] += jnp.sum(no, axis=0, keepdims=True)
    acc_h_ref[0:1, :] += jnp.sum(s, axis=0, keepdims=True)

    @pl.when(i == _NBLK - 1)
    def _globals():
        mean_no = acc_no_ref[0:1, :] * (1.0 / N)
        mean_em = (jnp.dot(acc_h_ref[0:1, :] * (1.0 / E), we2_ref[...],
                           preferred_element_type=jnp.float32) + be2_ref[...])
        g = g_ref[...]
        zg = (jnp.dot(g, wg1g_ref[...], preferred_element_type=jnp.float32)
              + jnp.dot(mean_no, wg1n_ref[...], preferred_element_type=jnp.float32)
              + jnp.dot(mean_em, wg1m_ref[...], preferred_element_type=jnp.float32)
              + bg1_ref[...])
        gh = zg / (1.0 + jnp.exp(zg * nbg_ref[...]))
        gout_ref[...] = g + jnp.dot(gh, wg2_ref[...],
                                    preferred_element_type=jnp.float32) + bg2_ref[...]


def _finish(sums, cnts, nf, g, we2, be2, wn1a, wn1b, bn1, nbn, wn2, bn2,
            wg1g, wg1n, wg1m, bg1, nbg, wg2, bg2):
    full = lambda shape: pl.BlockSpec(shape, lambda i: tuple(0 for _ in shape))
    row = lambda shape: pl.BlockSpec(shape, lambda i: (i, 0))
    return pl.pallas_call(
        _finish_body,
        grid=(_NBLK,),
        in_specs=[
            row((_RN, D)),
            row((_RN, D)),
            row((_RN, D)),
            full((1, D)),
            full((D, D)), full((1, D)),
            full((D, D)), full((D, D)), full((1, D)), full((1, D)),
            full((D, D)), full((1, D)),
            full((D, D)), full((D, D)), full((D, D)), full((1, D)), full((1, D)),
            full((D, D)), full((1, D)),
        ],
        out_specs=[row((_RN, D)), full((1, D))],
        out_shape=[jax.ShapeDtypeStruct((N, D), jnp.float32),
                   jax.ShapeDtypeStruct((1, D), jnp.float32)],
        scratch_shapes=[pltpu.VMEM((8, D), jnp.float32),
                        pltpu.VMEM((8, D), jnp.float32)],
    )(sums, cnts, nf, g, we2, be2, wn1a, wn1b, bn1, nbn, wn2, bn2,
      wg1g, wg1n, wg1m, bg1, nbg, wg2, bg2)


def kernel(node_features, edge_features, senders, receivers, globals_,
           We1, be1, betae, We2, be2,
           Wn1, bn1, betan, Wn2, bn2,
           Wg1, bg1, betag, Wg2, bg2):
    tabs = _node_prep(node_features, We1[DE:DE + D], We1[DE + D:],
                      be1[None, :])
    table = tabs.reshape(2 * N, D)
    ef1 = _edge_prep(edge_features, We1[:DE])

    r32 = receivers.astype(jnp.int32)
    sums = _sc_edge_kernel(
        table, ef1, senders.astype(jnp.int32), r32, -betae)
    cnts = _sc_count_kernel(r32)

    node_out, globals_out = _finish(
        sums, cnts,
        node_features, globals_,
        We2, be2[None, :],
        Wn1[:D], Wn1[D:], bn1[None, :], -betan[None, :], Wn2, bn2[None, :],
        Wg1[:D], Wg1[D:2 * D], Wg1[2 * D:], bg1[None, :], -betag[None, :],
        Wg2, bg2[None, :])

    return (node_out, edge_features, globals_out)
